# Initial kernel scaffold; baseline (speedup 1.0000x reference)
import jax
import jax.numpy as jnp
from jax.experimental import pallas as pl


def kernel(x, W, gamma, beta):
    raise NotImplementedError("write your pallas kernel here")


# speedup vs baseline: 13.4014x; 13.4014x over previous
import functools

import jax
import jax.numpy as jnp
from jax import lax
from jax.experimental import pallas as pl

B, N, C, K = 2, 4096, 64, 16
TILE = 256
NEG_BIG = -1e30


def _knn_maxpool_body(x_rows_ref, x_all_ref, feat_ref):
    xr = x_rows_ref[0]
    xa = x_all_ref[0]
    inner = lax.dot_general(xr, xa, (((1,), (1,)), ((), ())),
                            preferred_element_type=jnp.float32)
    xx_r = jnp.sum(xr * xr, axis=1, keepdims=True)
    xx_a = jnp.sum(xa * xa, axis=1).reshape(1, N)
    neg = 2.0 * inner - xx_r - xx_a
    col = lax.broadcasted_iota(jnp.int32, (TILE, N), 1)
    feat = jnp.full((TILE, C), NEG_BIG, dtype=jnp.float32)
    for _ in range(K):
        m = jnp.max(neg, axis=1, keepdims=True)
        amin = jnp.min(jnp.where(neg == m, col, N), axis=1,
                       keepdims=True)
        onehot = (col == amin).astype(jnp.float32)
        row = lax.dot_general(onehot, xa, (((1,), (0,)), ((), ())),
                              preferred_element_type=jnp.float32)
        feat = jnp.maximum(feat, row)
        neg = jnp.where(col == amin, NEG_BIG, neg)
    feat_ref[0] = feat


def _head_body(feat_ref, w_ref, gamma_ref, beta_ref, out_ref):
    feat = feat_ref[...]
    w = w_ref[...]
    y = lax.dot_general(feat, w, (((1,), (1,)), ((), ())),
                        preferred_element_type=jnp.float32)
    mean = jnp.mean(y, axis=0, keepdims=True)
    var = jnp.mean(y * y, axis=0, keepdims=True) - mean * mean
    yhat = (y - mean) * lax.rsqrt(var + 1e-5)
    y = yhat * gamma_ref[...] + beta_ref[...]
    out_ref[...] = jnp.where(y >= 0, y, 0.01 * y)


@jax.jit
def kernel(x, W, gamma, beta):
    feat = pl.pallas_call(
        _knn_maxpool_body,
        grid=(B, N // TILE),
        in_specs=[
            pl.BlockSpec((1, TILE, C), lambda b, i: (b, i, 0)),
            pl.BlockSpec((1, N, C), lambda b, i: (b, 0, 0)),
        ],
        out_specs=pl.BlockSpec((1, TILE, C), lambda b, i: (b, i, 0)),
        out_shape=jax.ShapeDtypeStruct((B, N, C), jnp.float32),
    )(x, x)
    out = pl.pallas_call(
        _head_body,
        out_shape=jax.ShapeDtypeStruct((B * N, C), jnp.float32),
    )(feat.reshape(B * N, C), W, gamma.reshape(1, C), beta.reshape(1, C))
    return out.reshape(B, N, C)


# speedup vs baseline: 13.4024x; 1.0001x over previous
import functools

import jax
import jax.numpy as jnp
from jax import lax
from jax.experimental import pallas as pl
from jax.experimental.pallas import tpu as pltpu
from jax.experimental.pallas import tpu_sc as plsc

B, N, C, K = 2, 4096, 64, 16
CP = 128
TILE = 256
NEG_BIG = -1e30

_info = plsc.get_sparse_core_info()
NC, NS, L = _info.num_cores, _info.num_subcores, _info.num_lanes
NW = NC * NS
PTS_PER_W = (B * N) // NW
CHUNK = 32
GATHER = 128


def _topk_idx_body(x_rows_ref, x_all_ref, idx_ref):
    b = pl.program_id(0)
    xr = x_rows_ref[0]
    xa = x_all_ref[0]
    inner = lax.dot_general(xr, xa, (((1,), (1,)), ((), ())),
                            preferred_element_type=jnp.float32)
    xx_r = jnp.sum(xr * xr, axis=1, keepdims=True)
    xx_a = jnp.sum(xa * xa, axis=1).reshape(1, N)
    neg = 2.0 * inner - xx_r - xx_a
    col = lax.broadcasted_iota(jnp.int32, (TILE, N), 1)
    picks = []
    for _ in range(K):
        m = jnp.max(neg, axis=1, keepdims=True)
        amin = jnp.min(jnp.where(neg == m, col, N), axis=1,
                       keepdims=True)
        picks.append(amin)
        neg = jnp.where(col == amin, NEG_BIG, neg)
    idx_ref[0] = jnp.concatenate(picks, axis=1) + b * N


def _head_body(feat_ref, w_ref, gamma_ref, beta_ref, out_ref):
    feat = feat_ref[...][:, :C]
    w = w_ref[...]
    y = lax.dot_general(feat, w, (((1,), (1,)), ((), ())),
                        preferred_element_type=jnp.float32)
    mean = jnp.mean(y, axis=0, keepdims=True)
    var = jnp.mean(y * y, axis=0, keepdims=True) - mean * mean
    yhat = (y - mean) * lax.rsqrt(var + 1e-5)
    y = yhat * gamma_ref[...] + beta_ref[...]
    out_ref[...] = jnp.where(y >= 0, y, 0.01 * y)


def _sc_gather_maxpool(x_hbm, idx_hbm, out_hbm, idx_v, rows_v, feat_v, sem):
    wid = lax.axis_index("s") * NC + lax.axis_index("c")
    base_pt = wid * PTS_PER_W

    for t in range(PTS_PER_W // CHUNK):
        pt0 = base_pt + t * CHUNK
        pltpu.sync_copy(idx_hbm.at[pl.ds(pt0 * K, CHUNK * K)], idx_v)
        copies = []
        for g in range(CHUNK * K // GATHER):
            copies.append(pltpu.async_copy(
                x_hbm.at[idx_v.at[pl.ds(g * GATHER, GATHER)]],
                rows_v.at[pl.ds(g * GATHER, GATHER), :], sem))
        for cp in copies:
            cp.wait()

        def body(p, _):
            for c4 in range(C // L):
                acc = rows_v[p * K, pl.ds(c4 * L, L)]
                for j in range(1, K):
                    acc = jnp.maximum(acc, rows_v[p * K + j, pl.ds(c4 * L, L)])
                feat_v[p, pl.ds(c4 * L, L)] = acc
            return 0

        lax.fori_loop(0, CHUNK, body, 0, unroll=False)
        pltpu.sync_copy(feat_v, out_hbm.at[pl.ds(pt0, CHUNK)])


_sc_gather = functools.partial(
    pl.kernel,
    mesh=plsc.VectorSubcoreMesh(core_axis_name="c", subcore_axis_name="s"),
    out_type=jax.ShapeDtypeStruct((B * N, CP), jnp.float32),
    scratch_types=[
        pltpu.VMEM((CHUNK * K,), jnp.int32),
        pltpu.VMEM((CHUNK * K, CP), jnp.float32),
        pltpu.VMEM((CHUNK, CP), jnp.float32),
        pltpu.SemaphoreType.DMA,
    ],
)(_sc_gather_maxpool)


@jax.jit
def kernel(x, W, gamma, beta):
    idx = pl.pallas_call(
        _topk_idx_body,
        grid=(B, N // TILE),
        in_specs=[
            pl.BlockSpec((1, TILE, C), lambda b, i: (b, i, 0)),
            pl.BlockSpec((1, N, C), lambda b, i: (b, 0, 0)),
        ],
        out_specs=pl.BlockSpec((1, TILE, K), lambda b, i: (b, i, 0)),
        out_shape=jax.ShapeDtypeStruct((B, N, K), jnp.int32),
    )(x, x)
    x_pad = jnp.pad(x.reshape(B * N, C), ((0, 0), (0, CP - C)))
    feat = _sc_gather(x_pad, idx.reshape(B * N * K))
    out = pl.pallas_call(
        _head_body,
        out_shape=jax.ShapeDtypeStruct((B * N, C), jnp.float32),
    )(feat, W, gamma.reshape(1, C), beta.reshape(1, C))
    return out.reshape(B, N, C)


# speedup vs baseline: 15.3264x; 1.1436x over previous
import functools

import jax
import jax.numpy as jnp
from jax import lax
from jax.experimental import pallas as pl
from jax.experimental.pallas import tpu as pltpu
from jax.experimental.pallas import tpu_sc as plsc

B, N, C, K = 2, 4096, 64, 16
CP = 128
TILE = 256
NEG_BIG = -1e30

_info = plsc.get_sparse_core_info()
NC, NS, L = _info.num_cores, _info.num_subcores, _info.num_lanes
NW = NC * NS
PTS_PER_W = (B * N) // NW
CHUNK = 32
GATHER = 128


def _topk_idx_body(x_rows_ref, x_all_ref, idx_ref):
    b = pl.program_id(0)
    xr = x_rows_ref[0]
    xa = x_all_ref[0]
    inner = lax.dot_general(xr, xa, (((1,), (1,)), ((), ())),
                            preferred_element_type=jnp.float32)
    xx_r = jnp.sum(xr * xr, axis=1, keepdims=True)
    xx_a = jnp.sum(xa * xa, axis=1).reshape(1, N)
    neg = 2.0 * inner - xx_r - xx_a
    col = lax.broadcasted_iota(jnp.int32, (TILE, N), 1)
    m = jnp.max(neg, axis=1, keepdims=True)
    picks = []
    for t in range(K):
        hit = neg == m
        amin = jnp.min(jnp.where(hit, col, N), axis=1,
                       keepdims=True)
        picks.append(amin)
        if t < K - 1:
            neg = jnp.where(hit, NEG_BIG, neg)
            m = jnp.max(neg, axis=1, keepdims=True)
    idx_ref[0] = jnp.concatenate(picks, axis=1) + b * N


def _head_body(feat_ref, w_ref, gamma_ref, beta_ref, out_ref):
    feat = feat_ref[...][:, :C]
    w = w_ref[...]
    y = lax.dot_general(feat, w, (((1,), (1,)), ((), ())),
                        preferred_element_type=jnp.float32)
    mean = jnp.mean(y, axis=0, keepdims=True)
    var = jnp.mean(y * y, axis=0, keepdims=True) - mean * mean
    yhat = (y - mean) * lax.rsqrt(var + 1e-5)
    y = yhat * gamma_ref[...] + beta_ref[...]
    out_ref[...] = jnp.where(y >= 0, y, 0.01 * y)


def _sc_gather_maxpool(x_hbm, idx_hbm, out_hbm, idx_v, rows_v, feat_v, sem):
    wid = lax.axis_index("s") * NC + lax.axis_index("c")
    base_pt = wid * PTS_PER_W

    for t in range(PTS_PER_W // CHUNK):
        pt0 = base_pt + t * CHUNK
        pltpu.sync_copy(idx_hbm.at[pl.ds(pt0 * K, CHUNK * K)], idx_v)
        copies = []
        for g in range(CHUNK * K // GATHER):
            copies.append(pltpu.async_copy(
                x_hbm.at[idx_v.at[pl.ds(g * GATHER, GATHER)]],
                rows_v.at[pl.ds(g * GATHER, GATHER), :], sem))
        for cp in copies:
            cp.wait()

        def body(p, _):
            for c4 in range(C // L):
                acc = rows_v[p * K, pl.ds(c4 * L, L)]
                for j in range(1, K):
                    acc = jnp.maximum(acc, rows_v[p * K + j, pl.ds(c4 * L, L)])
                feat_v[p, pl.ds(c4 * L, L)] = acc
            return 0

        lax.fori_loop(0, CHUNK, body, 0, unroll=False)
        pltpu.sync_copy(feat_v, out_hbm.at[pl.ds(pt0, CHUNK)])


_sc_gather = functools.partial(
    pl.kernel,
    mesh=plsc.VectorSubcoreMesh(core_axis_name="c", subcore_axis_name="s"),
    out_type=jax.ShapeDtypeStruct((B * N, CP), jnp.float32),
    scratch_types=[
        pltpu.VMEM((CHUNK * K,), jnp.int32),
        pltpu.VMEM((CHUNK * K, CP), jnp.float32),
        pltpu.VMEM((CHUNK, CP), jnp.float32),
        pltpu.SemaphoreType.DMA,
    ],
)(_sc_gather_maxpool)


@jax.jit
def kernel(x, W, gamma, beta):
    idx = pl.pallas_call(
        _topk_idx_body,
        grid=(B, N // TILE),
        in_specs=[
            pl.BlockSpec((1, TILE, C), lambda b, i: (b, i, 0)),
            pl.BlockSpec((1, N, C), lambda b, i: (b, 0, 0)),
        ],
        out_specs=pl.BlockSpec((1, TILE, K), lambda b, i: (b, i, 0)),
        out_shape=jax.ShapeDtypeStruct((B, N, K), jnp.int32),
    )(x, x)
    x_pad = jnp.pad(x.reshape(B * N, C), ((0, 0), (0, CP - C)))
    feat = _sc_gather(x_pad, idx.reshape(B * N * K))
    out = pl.pallas_call(
        _head_body,
        out_shape=jax.ShapeDtypeStruct((B * N, C), jnp.float32),
    )(feat, W, gamma.reshape(1, C), beta.reshape(1, C))
    return out.reshape(B, N, C)


# speedup vs baseline: 16.6404x; 1.0857x over previous
import functools

import jax
import jax.numpy as jnp
from jax import lax
from jax.experimental import pallas as pl
from jax.experimental.pallas import tpu as pltpu
from jax.experimental.pallas import tpu_sc as plsc

B, N, C, K = 2, 4096, 64, 16
CP = 128
TILE = 256
NEG_BIG = -1e30

_info = plsc.get_sparse_core_info()
NC, NS, L = _info.num_cores, _info.num_subcores, _info.num_lanes
NW = NC * NS
PTS_PER_W = (B * N) // NW
CHUNK = 32
GATHER = 128


def _topk_idx_body(x_rows_ref, x_all_ref, idx_ref):
    b = pl.program_id(0)
    xr = x_rows_ref[0]
    xa = x_all_ref[0]
    inner = lax.dot_general(xr, xa, (((1,), (1,)), ((), ())),
                            preferred_element_type=jnp.float32)
    xx_r = jnp.sum(xr * xr, axis=1, keepdims=True)
    xx_a = jnp.sum(xa * xa, axis=1).reshape(1, N)
    neg = 2.0 * inner - xx_r - xx_a
    colf = lax.broadcasted_iota(jnp.int32, (TILE, N), 1).astype(jnp.float32)
    BIGF = 1e9
    m = jnp.max(neg, axis=1, keepdims=True)
    picks = []
    for t in range(K):
        amin_f = jnp.min(jnp.where(neg == m, colf, BIGF), axis=1,
                         keepdims=True)
        picks.append(amin_f)
        if t < K - 1:
            m = jnp.max(jnp.where(neg < m, neg, NEG_BIG), axis=1,
                        keepdims=True)
    idx_f = jnp.minimum(jnp.concatenate(picks, axis=1), float(N - 1))
    idx_ref[0] = idx_f.astype(jnp.int32) + b * N


def _head_body(feat_ref, w_ref, gamma_ref, beta_ref, out_ref):
    feat = feat_ref[...][:, :C]
    w = w_ref[...]
    y = lax.dot_general(feat, w, (((1,), (1,)), ((), ())),
                        preferred_element_type=jnp.float32)
    mean = jnp.mean(y, axis=0, keepdims=True)
    var = jnp.mean(y * y, axis=0, keepdims=True) - mean * mean
    yhat = (y - mean) * lax.rsqrt(var + 1e-5)
    y = yhat * gamma_ref[...] + beta_ref[...]
    out_ref[...] = jnp.where(y >= 0, y, 0.01 * y)


def _sc_gather_maxpool(x_hbm, idx_hbm, out_hbm, idx_v, rows_v, feat_v, sem):
    wid = lax.axis_index("s") * NC + lax.axis_index("c")
    base_pt = wid * PTS_PER_W

    for t in range(PTS_PER_W // CHUNK):
        pt0 = base_pt + t * CHUNK
        pltpu.sync_copy(idx_hbm.at[pl.ds(pt0 * K, CHUNK * K)], idx_v)
        copies = []
        for g in range(CHUNK * K // GATHER):
            copies.append(pltpu.async_copy(
                x_hbm.at[idx_v.at[pl.ds(g * GATHER, GATHER)]],
                rows_v.at[pl.ds(g * GATHER, GATHER), :], sem))
        for cp in copies:
            cp.wait()

        def body(p, _):
            for c4 in range(C // L):
                acc = rows_v[p * K, pl.ds(c4 * L, L)]
                for j in range(1, K):
                    acc = jnp.maximum(acc, rows_v[p * K + j, pl.ds(c4 * L, L)])
                feat_v[p, pl.ds(c4 * L, L)] = acc
            return 0

        lax.fori_loop(0, CHUNK, body, 0, unroll=False)
        pltpu.sync_copy(feat_v, out_hbm.at[pl.ds(pt0, CHUNK)])


_sc_gather = functools.partial(
    pl.kernel,
    mesh=plsc.VectorSubcoreMesh(core_axis_name="c", subcore_axis_name="s"),
    out_type=jax.ShapeDtypeStruct((B * N, CP), jnp.float32),
    scratch_types=[
        pltpu.VMEM((CHUNK * K,), jnp.int32),
        pltpu.VMEM((CHUNK * K, CP), jnp.float32),
        pltpu.VMEM((CHUNK, CP), jnp.float32),
        pltpu.SemaphoreType.DMA,
    ],
)(_sc_gather_maxpool)


@jax.jit
def kernel(x, W, gamma, beta):
    idx = pl.pallas_call(
        _topk_idx_body,
        grid=(B, N // TILE),
        in_specs=[
            pl.BlockSpec((1, TILE, C), lambda b, i: (b, i, 0)),
            pl.BlockSpec((1, N, C), lambda b, i: (b, 0, 0)),
        ],
        out_specs=pl.BlockSpec((1, TILE, K), lambda b, i: (b, i, 0)),
        out_shape=jax.ShapeDtypeStruct((B, N, K), jnp.int32),
    )(x, x)
    x_pad = jnp.pad(x.reshape(B * N, C), ((0, 0), (0, CP - C)))
    feat = _sc_gather(x_pad, idx.reshape(B * N * K))
    out = pl.pallas_call(
        _head_body,
        out_shape=jax.ShapeDtypeStruct((B * N, C), jnp.float32),
    )(feat, W, gamma.reshape(1, C), beta.reshape(1, C))
    return out.reshape(B, N, C)


# speedup vs baseline: 17.0644x; 1.0255x over previous
import functools

import jax
import jax.numpy as jnp
from jax import lax
from jax.experimental import pallas as pl
from jax.experimental.pallas import tpu as pltpu
from jax.experimental.pallas import tpu_sc as plsc

B, N, C, K = 2, 4096, 64, 16
CP = 128
TILE = 512
NEG_BIG = -1e30

_info = plsc.get_sparse_core_info()
NC, NS, L = _info.num_cores, _info.num_subcores, _info.num_lanes
NW = NC * NS
PTS_PER_W = (B * N) // NW
CHUNK = 32
GATHER = 128


def _topk_idx_body(x_rows_ref, x_all_ref, idx_ref):
    b = pl.program_id(0)
    xr = x_rows_ref[0]
    xa = x_all_ref[0]
    inner = lax.dot_general(xr, xa, (((1,), (1,)), ((), ())),
                            preferred_element_type=jnp.float32)
    xx_r = jnp.sum(xr * xr, axis=1, keepdims=True)
    xx_a = jnp.sum(xa * xa, axis=1).reshape(1, N)
    neg = 2.0 * inner - xx_r - xx_a
    colf = lax.broadcasted_iota(jnp.int32, (TILE, N), 1).astype(jnp.float32)
    BIGF = 1e9
    m = jnp.max(neg, axis=1, keepdims=True)
    picks = []
    for t in range(K):
        amin_f = jnp.min(jnp.where(neg == m, colf, BIGF), axis=1,
                         keepdims=True)
        picks.append(amin_f)
        if t < K - 1:
            m = jnp.max(jnp.where(neg < m, neg, NEG_BIG), axis=1,
                        keepdims=True)
    idx_f = jnp.minimum(jnp.concatenate(picks, axis=1), float(N - 1))
    idx_ref[0] = idx_f.astype(jnp.int32) + b * N


def _head_body(feat_ref, w_ref, gamma_ref, beta_ref, out_ref):
    feat = feat_ref[...][:, :C]
    w = w_ref[...]
    y = lax.dot_general(feat, w, (((1,), (1,)), ((), ())),
                        preferred_element_type=jnp.float32)
    mean = jnp.mean(y, axis=0, keepdims=True)
    var = jnp.mean(y * y, axis=0, keepdims=True) - mean * mean
    yhat = (y - mean) * lax.rsqrt(var + 1e-5)
    y = yhat * gamma_ref[...] + beta_ref[...]
    out_ref[...] = jnp.where(y >= 0, y, 0.01 * y)


def _sc_gather_maxpool(x_hbm, idx_hbm, out_hbm, idx_v, rows_v, feat_v, sem):
    wid = lax.axis_index("s") * NC + lax.axis_index("c")
    base_pt = wid * PTS_PER_W

    for t in range(PTS_PER_W // CHUNK):
        pt0 = base_pt + t * CHUNK
        pltpu.sync_copy(idx_hbm.at[pl.ds(pt0 * K, CHUNK * K)], idx_v)
        copies = []
        for g in range(CHUNK * K // GATHER):
            copies.append(pltpu.async_copy(
                x_hbm.at[idx_v.at[pl.ds(g * GATHER, GATHER)]],
                rows_v.at[pl.ds(g * GATHER, GATHER), :], sem))
        for cp in copies:
            cp.wait()

        def body(p, _):
            for c4 in range(C // L):
                acc = rows_v[p * K, pl.ds(c4 * L, L)]
                for j in range(1, K):
                    acc = jnp.maximum(acc, rows_v[p * K + j, pl.ds(c4 * L, L)])
                feat_v[p, pl.ds(c4 * L, L)] = acc
            return 0

        lax.fori_loop(0, CHUNK, body, 0, unroll=False)
        pltpu.sync_copy(feat_v, out_hbm.at[pl.ds(pt0, CHUNK)])


_sc_gather = functools.partial(
    pl.kernel,
    mesh=plsc.VectorSubcoreMesh(core_axis_name="c", subcore_axis_name="s"),
    out_type=jax.ShapeDtypeStruct((B * N, CP), jnp.float32),
    scratch_types=[
        pltpu.VMEM((CHUNK * K,), jnp.int32),
        pltpu.VMEM((CHUNK * K, CP), jnp.float32),
        pltpu.VMEM((CHUNK, CP), jnp.float32),
        pltpu.SemaphoreType.DMA,
    ],
)(_sc_gather_maxpool)


@jax.jit
def kernel(x, W, gamma, beta):
    idx = pl.pallas_call(
        _topk_idx_body,
        grid=(B, N // TILE),
        in_specs=[
            pl.BlockSpec((1, TILE, C), lambda b, i: (b, i, 0)),
            pl.BlockSpec((1, N, C), lambda b, i: (b, 0, 0)),
        ],
        out_specs=pl.BlockSpec((1, TILE, K), lambda b, i: (b, i, 0)),
        out_shape=jax.ShapeDtypeStruct((B, N, K), jnp.int32),
    )(x, x)
    x_pad = jnp.pad(x.reshape(B * N, C), ((0, 0), (0, CP - C)))
    feat = _sc_gather(x_pad, idx.reshape(B * N * K))
    out = pl.pallas_call(
        _head_body,
        out_shape=jax.ShapeDtypeStruct((B * N, C), jnp.float32),
    )(feat, W, gamma.reshape(1, C), beta.reshape(1, C))
    return out.reshape(B, N, C)
